# Initial kernel scaffold; baseline (speedup 1.0000x reference)
import jax
import jax.numpy as jnp
from jax.experimental import pallas as pl


def kernel(tokens, W):
    raise NotImplementedError("write your pallas kernel here")


# speedup vs baseline: 1.1451x; 1.1451x over previous
import functools

import jax
import jax.numpy as jnp
from jax import lax
from jax.experimental import pallas as pl
from jax.experimental.pallas import tpu as pltpu
from jax.experimental.pallas import tpu_sc as plsc

PAD = 1
B_ROWS = 4
SEQ = 2048
EMB = 1024
TOKENS = B_ROWS * SEQ

NC = 2
NS = 16
NW = NC * NS
CHUNK = TOKENS // NW
CHUNKS_PER_ROW = SEQ // CHUNK
SUB = 32
NSUB = CHUNK // SUB
LANES = 16


def _sc_kernel(tok_hbm, w_hbm, out_hbm, tok_v, pos_v, shf_v, buf0, buf1,
               gsem0, gsem1, osem0, osem1):
  wid = lax.axis_index("s") * NC + lax.axis_index("c")
  row = wid // CHUNKS_PER_ROW
  chunk = wid % CHUNKS_PER_ROW
  t0 = chunk * CHUNK
  base = row * SEQ + t0

  pltpu.sync_copy(tok_hbm.at[pl.ds(row * SEQ, SEQ)], tok_v)

  shf_v[pl.ds(0, LANES)] = jnp.zeros((LANES,), jnp.int32)

  def cumsum16(x):
    for k in (1, 2, 4, 8):
      shf_v[pl.ds(LANES, LANES)] = x
      x = x + shf_v[pl.ds(LANES - k, LANES)]
    return x

  def mask16(v):
    return jnp.minimum(jnp.abs(v - PAD), 1)

  def count_body(jj, acc):
    cs = cumsum16(mask16(tok_v[pl.ds(jj * LANES, LANES)]))
    return acc + cs[LANES - 1]

  carry = lax.fori_loop(0, t0 // LANES, count_body, jnp.int32(0))

  for j in range(CHUNK // LANES):
    v = tok_v[pl.ds(t0 + j * LANES, LANES)]
    mi = mask16(v)
    cs = cumsum16(mi)
    pos_v[j // 2, pl.ds((j % 2) * LANES, LANES)] = (carry + cs) * mi + 1
    carry = carry + cs[LANES - 1]

  bufs = (buf0, buf1)
  gsems = (gsem0, gsem1)
  osems = (osem0, osem1)

  def gather(i, b):
    return pltpu.async_copy(w_hbm.at[pos_v.at[i]], bufs[b], gsems[b])

  def put(i, b):
    return pltpu.async_copy(bufs[b], out_hbm.at[pl.ds(base + i * SUB, SUB)],
                            osems[b])

  gh = [None, None]
  oh = [None, None]
  gh[0] = gather(0, 0)
  for i in range(NSUB):
    b = i % 2
    gh[b].wait()
    if i + 1 < NSUB:
      nb = 1 - b
      if i >= 1:
        oh[nb].wait()
      gh[nb] = gather(i + 1, nb)
    oh[b] = put(i, b)
  oh[(NSUB - 1) % 2].wait()


@jax.jit
def _lookup(tokens_flat, w):
  mesh = plsc.VectorSubcoreMesh(core_axis_name="c", subcore_axis_name="s")
  k = functools.partial(
      pl.kernel,
      mesh=mesh,
      out_type=jax.ShapeDtypeStruct((TOKENS, EMB), jnp.float32),
      scratch_types=[
          pltpu.VMEM((SEQ,), jnp.int32),
          pltpu.VMEM((NSUB, SUB), jnp.int32),
          pltpu.VMEM((2 * LANES,), jnp.int32),
          pltpu.VMEM((SUB, EMB), jnp.float32),
          pltpu.VMEM((SUB, EMB), jnp.float32),
          pltpu.SemaphoreType.DMA,
          pltpu.SemaphoreType.DMA,
          pltpu.SemaphoreType.DMA,
          pltpu.SemaphoreType.DMA,
      ],
  )(_sc_kernel)
  return k(tokens_flat, w)


def kernel(tokens, W):
  tokens_flat = tokens.astype(jnp.int32).reshape(TOKENS)
  out = _lookup(tokens_flat, W)
  return out.reshape(B_ROWS, SEQ, EMB)


# speedup vs baseline: 1.1547x; 1.0084x over previous
import functools

import jax
import jax.numpy as jnp
from jax import lax
from jax.experimental import pallas as pl
from jax.experimental.pallas import tpu as pltpu
from jax.experimental.pallas import tpu_sc as plsc

PAD = 1
B_ROWS = 4
SEQ = 2048
EMB = 1024
TOKENS = B_ROWS * SEQ

NC = 2
NS = 16
NW = NC * NS
CHUNK = TOKENS // NW
CHUNKS_PER_ROW = SEQ // CHUNK
SUB = 32
NSUB = CHUNK // SUB
NBUF = 3
LANES = 16


def _sc_kernel(tok_hbm, w_hbm, out_hbm, tok_v, pos_v, shf_v, buf0, buf1,
               buf2, gsem0, gsem1, gsem2, osem0, osem1, osem2):
  wid = lax.axis_index("s") * NC + lax.axis_index("c")
  row = wid // CHUNKS_PER_ROW
  chunk = wid % CHUNKS_PER_ROW
  t0 = chunk * CHUNK
  base = row * SEQ + t0

  pltpu.sync_copy(tok_hbm.at[pl.ds(row * SEQ, SEQ)], tok_v)

  shf_v[pl.ds(0, LANES)] = jnp.zeros((LANES,), jnp.int32)

  def cumsum16(x):
    for k in (1, 2, 4, 8):
      shf_v[pl.ds(LANES, LANES)] = x
      x = x + shf_v[pl.ds(LANES - k, LANES)]
    return x

  def mask16(v):
    return jnp.minimum(jnp.abs(v - PAD), 1)

  def count_body(jj, acc):
    return acc + mask16(tok_v[pl.ds(jj * LANES, LANES)])

  acc = lax.fori_loop(0, t0 // LANES, count_body,
                      jnp.zeros((LANES,), jnp.int32))
  carry = cumsum16(acc)[LANES - 1]

  for j in range(CHUNK // LANES):
    v = tok_v[pl.ds(t0 + j * LANES, LANES)]
    mi = mask16(v)
    cs = cumsum16(mi)
    pos_v[j // 2, pl.ds((j % 2) * LANES, LANES)] = (carry + cs) * mi + 1
    carry = carry + cs[LANES - 1]

  bufs = (buf0, buf1, buf2)
  gsems = (gsem0, gsem1, gsem2)
  osems = (osem0, osem1, osem2)

  def gather(i):
    b = i % NBUF
    return pltpu.async_copy(w_hbm.at[pos_v.at[i]], bufs[b], gsems[b])

  def put(i):
    b = i % NBUF
    return pltpu.async_copy(bufs[b], out_hbm.at[pl.ds(base + i * SUB, SUB)],
                            osems[b])

  gh = [None] * NSUB
  oh = [None] * NSUB
  waited = set()
  gh[0] = gather(0)
  for i in range(NSUB):
    gh[i].wait()
    if i + 1 < NSUB:
      if i + 1 >= NBUF:
        oh[i + 1 - NBUF].wait()
        waited.add(i + 1 - NBUF)
      gh[i + 1] = gather(i + 1)
    oh[i] = put(i)
  for i in range(NSUB):
    if i not in waited:
      oh[i].wait()


@jax.jit
def _lookup(tokens_flat, w):
  mesh = plsc.VectorSubcoreMesh(core_axis_name="c", subcore_axis_name="s")
  k = functools.partial(
      pl.kernel,
      mesh=mesh,
      out_type=jax.ShapeDtypeStruct((TOKENS, EMB), jnp.float32),
      scratch_types=[
          pltpu.VMEM((SEQ,), jnp.int32),
          pltpu.VMEM((NSUB, SUB), jnp.int32),
          pltpu.VMEM((2 * LANES,), jnp.int32),
          pltpu.VMEM((SUB, EMB), jnp.float32),
          pltpu.VMEM((SUB, EMB), jnp.float32),
          pltpu.VMEM((SUB, EMB), jnp.float32),
          pltpu.SemaphoreType.DMA,
          pltpu.SemaphoreType.DMA,
          pltpu.SemaphoreType.DMA,
          pltpu.SemaphoreType.DMA,
          pltpu.SemaphoreType.DMA,
          pltpu.SemaphoreType.DMA,
      ],
  )(_sc_kernel)
  return k(tokens_flat, w)


def kernel(tokens, W):
  tokens_flat = tokens.astype(jnp.int32).reshape(TOKENS)
  out = _lookup(tokens_flat, W)
  return out.reshape(B_ROWS, SEQ, EMB)


# speedup vs baseline: 1.1728x; 1.0156x over previous
import functools

import jax
import jax.numpy as jnp
from jax import lax
from jax.experimental import pallas as pl
from jax.experimental.pallas import tpu as pltpu
from jax.experimental.pallas import tpu_sc as plsc

PAD = 1
B_ROWS = 4
SEQ = 2048
EMB = 1024
TOKENS = B_ROWS * SEQ

NC = 2
NS = 16
NW = NC * NS
CHUNK = TOKENS // NW
CHUNKS_PER_ROW = SEQ // CHUNK
SUB = 32
NSUB = CHUNK // SUB
NBUF = 3
LANES = 16


def _sc_kernel(tok_hbm, w_hbm, out_hbm, tok_v, pos_v, shf_v, buf0, buf1,
               buf2, gsem0, gsem1, gsem2, osem0, osem1, osem2):
  wid = lax.axis_index("s") * NC + lax.axis_index("c")
  row = wid // CHUNKS_PER_ROW
  chunk = wid % CHUNKS_PER_ROW
  t0 = chunk * CHUNK
  base = row * SEQ + t0

  pltpu.sync_copy(tok_hbm.at[pl.ds(row * SEQ, SEQ)], tok_v)

  shf_v[pl.ds(0, LANES)] = jnp.zeros((LANES,), jnp.int32)

  def cumsum16(x):
    for k in (1, 2, 4, 8):
      shf_v[pl.ds(LANES, LANES)] = x
      x = x + shf_v[pl.ds(LANES - k, LANES)]
    return x

  def mask16(v):
    return jnp.minimum(jnp.abs(v - PAD), 1)

  def count_body(jj, acc):
    return acc + mask16(tok_v[pl.ds(jj * LANES, LANES)])

  acc = lax.fori_loop(0, t0 // LANES, count_body,
                      jnp.zeros((LANES,), jnp.int32))
  carry = cumsum16(acc)[LANES - 1]

  bufs = (buf0, buf1, buf2)
  gsems = (gsem0, gsem1, gsem2)
  osems = (osem0, osem1, osem2)

  def gather(i):
    b = i % NBUF
    return pltpu.async_copy(w_hbm.at[pos_v.at[i]], bufs[b], gsems[b])

  def put(i):
    b = i % NBUF
    return pltpu.async_copy(bufs[b], out_hbm.at[pl.ds(base + i * SUB, SUB)],
                            osems[b])

  gh = [None] * NSUB
  oh = [None] * NSUB
  waited = set()
  GPB = SUB // LANES
  for i in range(NSUB):
    for g in range(GPB):
      j = i * GPB + g
      v = tok_v[pl.ds(t0 + j * LANES, LANES)]
      mi = mask16(v)
      cs = cumsum16(mi)
      pos_v[i, pl.ds(g * LANES, LANES)] = (carry + cs) * mi + 1
      carry = carry + cs[LANES - 1]
    if i >= NBUF:
      oh[i - NBUF].wait()
      waited.add(i - NBUF)
    gh[i] = gather(i)
    if i >= 1:
      gh[i - 1].wait()
      oh[i - 1] = put(i - 1)
  gh[NSUB - 1].wait()
  oh[NSUB - 1] = put(NSUB - 1)
  for i in range(NSUB):
    if i not in waited:
      oh[i].wait()


@jax.jit
def _lookup(tokens_flat, w):
  mesh = plsc.VectorSubcoreMesh(core_axis_name="c", subcore_axis_name="s")
  k = functools.partial(
      pl.kernel,
      mesh=mesh,
      out_type=jax.ShapeDtypeStruct((TOKENS, EMB), jnp.float32),
      scratch_types=[
          pltpu.VMEM((SEQ,), jnp.int32),
          pltpu.VMEM((NSUB, SUB), jnp.int32),
          pltpu.VMEM((2 * LANES,), jnp.int32),
          pltpu.VMEM((SUB, EMB), jnp.float32),
          pltpu.VMEM((SUB, EMB), jnp.float32),
          pltpu.VMEM((SUB, EMB), jnp.float32),
          pltpu.SemaphoreType.DMA,
          pltpu.SemaphoreType.DMA,
          pltpu.SemaphoreType.DMA,
          pltpu.SemaphoreType.DMA,
          pltpu.SemaphoreType.DMA,
          pltpu.SemaphoreType.DMA,
      ],
  )(_sc_kernel)
  return k(tokens_flat, w)


def kernel(tokens, W):
  tokens_flat = tokens.astype(jnp.int32).reshape(TOKENS)
  out = _lookup(tokens_flat, W)
  return out.reshape(B_ROWS, SEQ, EMB)
